# baseline (speedup 1.0000x reference)
import functools

import numpy as np
import jax
import jax.numpy as jnp
from jax import lax
from jax.experimental import pallas as pl
from jax.experimental.pallas import tpu as pltpu
from jax.experimental.pallas import tpu_sc as plsc

_NC, _NS = 2, 16
_NW = _NC * _NS
_CHUNK = 128
_NB = 5
_NSLOT = 2
_LOOKAHEAD = 3


def _pos_encoding_np(max_seq_len, embed_dim, n=10000.0):
    position = np.arange(max_seq_len, dtype=np.float32)[:, None]
    division_term = np.exp(
        np.arange(0, embed_dim, 2, dtype=np.float32) * (-np.log(n) / embed_dim))
    pe = np.zeros((max_seq_len, embed_dim), dtype=np.float32)
    pe[:, 0::2] = np.sin(position * division_term)
    pe[:, 1::2] = np.cos(position * division_term)
    return pe


@functools.partial(jax.jit, static_argnames=("seq_len",))
def _sc_embed(table, idx3d, pe, *, seq_len):
    n_chunks = idx3d.shape[1]
    rows_per_w = n_chunks * _CHUNK
    total_rows = _NW * rows_per_w
    d = table.shape[1]

    mesh = plsc.VectorSubcoreMesh(core_axis_name="c", subcore_axis_name="s",
                                  num_cores=_NC, num_subcores=_NS)

    scratch = (
        [pltpu.VMEM((n_chunks, _CHUNK), jnp.int32)]
        + [pltpu.VMEM((_CHUNK, d), jnp.float32) for _ in range(_NB)]
        + [pltpu.SemaphoreType.DMA for _ in range(2 * _NB + _NSLOT)]
        + [pltpu.VMEM_SHARED((seq_len, d), jnp.float32),
           pltpu.VMEM_SHARED((_NS, _NSLOT, _CHUNK, d), jnp.float32)]
    )

    @functools.partial(
        pl.kernel,
        out_type=jax.ShapeDtypeStruct((total_rows, d), jnp.float32),
        mesh=mesh,
        scratch_types=scratch,
    )
    def body(table_hbm, idx_hbm, pe_hbm, out_hbm, idx_v, *rest):
        bufs = rest[:_NB]
        gsems = rest[_NB:2 * _NB]
        xsems = rest[2 * _NB:3 * _NB]
        wsems = rest[3 * _NB:3 * _NB + _NSLOT]
        pe_sh = rest[3 * _NB + _NSLOT]
        sp_slots = rest[3 * _NB + _NSLOT + 1]

        sid = lax.axis_index("s")
        w = sid * _NC + lax.axis_index("c")
        pltpu.sync_copy(idx_hbm.at[w], idx_v)

        @pl.when(sid == 0)
        def _stage_pe():
            off = 0
            while off < seq_len:
                n = min(_CHUNK, seq_len - off)
                sl = pl.ds(off, n)
                pltpu.sync_copy(pe_hbm.at[sl], bufs[0].at[pl.ds(0, n)])
                pltpu.sync_copy(bufs[0].at[pl.ds(0, n)], pe_sh.at[sl])
                off += n

        plsc.subcore_barrier()
        row0 = w * rows_per_w

        pe_descs = [None] * _NB
        gath = [None] * n_chunks
        xbar = [None] * n_chunks
        wr = [None] * n_chunks

        my_slots = sp_slots.at[sid]

        def free_and_pe_init(j):
            if not (0 <= j < n_chunks):
                return
            jj = j - _NB
            if jj >= 0 and xbar[jj] is not None:
                xbar[jj].wait()
                xbar[jj] = None
            b = j % _NB
            descs = []
            off = (j * _CHUNK) % seq_len
            done = 0
            while done < _CHUNK:
                n = min(_CHUNK - done, seq_len - off)
                descs.append(pltpu.async_copy(
                    pe_sh.at[pl.ds(off, n)],
                    bufs[b].at[pl.ds(done, n)], gsems[b]))
                done += n
                off = (off + n) % seq_len
            pe_descs[b] = descs

        def launch_gather(j):
            if not (0 <= j < n_chunks):
                return
            b = j % _NB
            for dsc in pe_descs[b]:
                dsc.wait()
            gath[j] = pltpu.async_copy(
                table_hbm.at[idx_v.at[j]], bufs[b], gsems[b], add=True)

        def launch_write(j):
            if not (0 <= j < n_chunks) or xbar[j] is None:
                return
            xbar[j].wait()
            xbar[j] = None
            wr[j] = pltpu.async_copy(
                my_slots.at[j % _NSLOT],
                out_hbm.at[pl.ds(row0 + j * _CHUNK, _CHUNK)],
                wsems[j % _NSLOT])

        for j in range(_LOOKAHEAD + 1):
            free_and_pe_init(j)
        for j in range(_LOOKAHEAD):
            launch_gather(j)
        for k in range(n_chunks):
            free_and_pe_init(k + _LOOKAHEAD + 1)
            launch_gather(k + _LOOKAHEAD)
            gath[k].wait()
            gath[k] = None
            if k - _NSLOT >= 0 and wr[k - _NSLOT] is not None:
                wr[k - _NSLOT].wait()
                wr[k - _NSLOT] = None
            xbar[k] = pltpu.async_copy(
                bufs[k % _NB], my_slots.at[k % _NSLOT], xsems[k % _NB])
            launch_write(k - 1)
        launch_write(n_chunks - 1)
        for j in range(n_chunks):
            if wr[j] is not None:
                wr[j].wait()

    return body(table, idx3d, pe)


def kernel(x, embedding_weight):
    b, s = x.shape
    d = embedding_weight.shape[1]
    total = b * s
    assert total % (_NW * _CHUNK) == 0 and s % 8 == 0
    n_chunks = total // (_NW * _CHUNK)
    idx3d = x.astype(jnp.int32).reshape(_NW, n_chunks, _CHUNK)
    pe = jnp.asarray(_pos_encoding_np(s, d))
    out = _sc_embed(embedding_weight, idx3d, pe, seq_len=s)
    return out.reshape(b, s, d)


# speedup vs baseline: 1.3206x; 1.3206x over previous
import functools

import numpy as np
import jax
import jax.numpy as jnp
from jax import lax
from jax.experimental import pallas as pl
from jax.experimental.pallas import tpu as pltpu
from jax.experimental.pallas import tpu_sc as plsc

_NC, _NS = 2, 16
_NW = _NC * _NS
_CHUNK = 128
_NB = 7
_LOOKAHEAD = 4


def _pos_encoding_np(max_seq_len, embed_dim, n=10000.0):
    position = np.arange(max_seq_len, dtype=np.float32)[:, None]
    division_term = np.exp(
        np.arange(0, embed_dim, 2, dtype=np.float32) * (-np.log(n) / embed_dim))
    pe = np.zeros((max_seq_len, embed_dim), dtype=np.float32)
    pe[:, 0::2] = np.sin(position * division_term)
    pe[:, 1::2] = np.cos(position * division_term)
    return pe


@functools.partial(jax.jit, static_argnames=("seq_len",))
def _sc_embed(table, idx3d, pe, *, seq_len):
    n_chunks = idx3d.shape[1]
    rows_per_w = n_chunks * _CHUNK
    total_rows = _NW * rows_per_w
    d = table.shape[1]

    mesh = plsc.VectorSubcoreMesh(core_axis_name="c", subcore_axis_name="s",
                                  num_cores=_NC, num_subcores=_NS)

    scratch = (
        [pltpu.VMEM((n_chunks, _CHUNK), jnp.int32)]
        + [pltpu.VMEM((_CHUNK, d), jnp.float32) for _ in range(_NB)]
        + [pltpu.SemaphoreType.DMA for _ in range(2 * _NB)]
        + [pltpu.VMEM_SHARED((seq_len, d), jnp.float32)]
    )

    @functools.partial(
        pl.kernel,
        out_type=jax.ShapeDtypeStruct((total_rows, d), jnp.float32),
        mesh=mesh,
        scratch_types=scratch,
    )
    def body(table_hbm, idx_hbm, pe_hbm, out_hbm, idx_v, *rest):
        bufs = rest[:_NB]
        gsems = rest[_NB:2 * _NB]
        osems = rest[2 * _NB:3 * _NB]
        pe_sh = rest[3 * _NB]

        sid = lax.axis_index("s")
        w = sid * _NC + lax.axis_index("c")
        pltpu.sync_copy(idx_hbm.at[w], idx_v)

        @pl.when(sid == 0)
        def _stage_pe():
            off = 0
            while off < seq_len:
                n = min(_CHUNK, seq_len - off)
                sl = pl.ds(off, n)
                pltpu.sync_copy(pe_hbm.at[sl], bufs[0].at[pl.ds(0, n)])
                pltpu.sync_copy(bufs[0].at[pl.ds(0, n)], pe_sh.at[sl])
                off += n

        plsc.subcore_barrier()
        row0 = w * rows_per_w

        pe_descs = [None] * _NB
        gath = [None] * n_chunks
        scat = [None] * n_chunks

        def free_and_pe_init(j):
            if not (0 <= j < n_chunks):
                return
            jj = j - _NB
            if jj >= 0:
                scat[jj].wait()
                scat[jj] = None
            b = j % _NB
            descs = []
            off = (j * _CHUNK) % seq_len
            done = 0
            while done < _CHUNK:
                n = min(_CHUNK - done, seq_len - off)
                descs.append(pltpu.async_copy(
                    pe_sh.at[pl.ds(off, n)],
                    bufs[b].at[pl.ds(done, n)], gsems[b]))
                done += n
                off = (off + n) % seq_len
            pe_descs[b] = descs

        def launch_gather(j):
            if not (0 <= j < n_chunks):
                return
            b = j % _NB
            for dsc in pe_descs[b]:
                dsc.wait()
            gath[j] = pltpu.async_copy(
                table_hbm.at[idx_v.at[j]], bufs[b], gsems[b], add=True)

        for j in range(_LOOKAHEAD + 1):
            free_and_pe_init(j)
        for j in range(_LOOKAHEAD):
            launch_gather(j)
        for k in range(n_chunks):
            free_and_pe_init(k + _LOOKAHEAD + 1)
            launch_gather(k + _LOOKAHEAD)
            gath[k].wait()
            gath[k] = None
            scat[k] = pltpu.async_copy(
                bufs[k % _NB], out_hbm.at[pl.ds(row0 + k * _CHUNK, _CHUNK)],
                osems[k % _NB])
        for j in range(n_chunks):
            if scat[j] is not None:
                scat[j].wait()

    return body(table, idx3d, pe)


def kernel(x, embedding_weight):
    b, s = x.shape
    d = embedding_weight.shape[1]
    total = b * s
    assert total % (_NW * _CHUNK) == 0 and s % 8 == 0
    n_chunks = total // (_NW * _CHUNK)
    idx3d = x.astype(jnp.int32).reshape(_NW, n_chunks, _CHUNK)
    pe = jnp.asarray(_pos_encoding_np(s, d))
    out = _sc_embed(embedding_weight, idx3d, pe, seq_len=s)
    return out.reshape(b, s, d)
